# Initial kernel scaffold; baseline (speedup 1.0000x reference)
import jax
import jax.numpy as jnp
from jax.experimental import pallas as pl


def kernel(x, edge_index, kernel_idx, W, gamma, beta):
    raise NotImplementedError("write your pallas kernel here")


# speedup vs baseline: 5.3401x; 5.3401x over previous
import functools

import jax
import jax.numpy as jnp
from jax import lax
from jax.experimental import pallas as pl
from jax.experimental.pallas import tpu as pltpu
from jax.experimental.pallas import tpu_sc as plsc

N = 10000
E = 320000
CIN = 128
COUT = 128
K = 27
EPS = 1e-5

NPAD = 10240
NC, NS = 2, 16
NW = NC * NS
EPT = E // NW
CH = 80
NCHUNK = EPT // CH
ROWS_PER_SUB = NPAD // NS
ZB = 40


def _z_body(x_ref, w_ref, z_ref):
    z_ref[0] = jnp.dot(x_ref[...], w_ref[0], preferred_element_type=jnp.float32)


def _z_matmul(x, W):
    nb = 10
    bn = N // nb
    return pl.pallas_call(
        _z_body,
        grid=(nb, K),
        in_specs=[
            pl.BlockSpec((bn, CIN), lambda i, k: (i, 0)),
            pl.BlockSpec((1, CIN, COUT), lambda i, k: (k, 0, 0)),
        ],
        out_specs=pl.BlockSpec((1, bn, COUT), lambda i, k: (k, i, 0)),
        out_shape=jax.ShapeDtypeStruct((K, N, COUT), jnp.float32),
    )(x, W)


def _sc_body(z_hbm, src_hbm, dst_hbm, kidx_hbm, out_hbm,
             src_v, dst_v, kidx_v, row_i, dst_i, rows_v, zbuf, hsh, sem):
    c = lax.axis_index("c")
    s = lax.axis_index("s")
    wid = s * NC + c

    zeros = jnp.zeros((16,), jnp.float32)

    def _zfill(t, carry):
        r = t // (COUT // 16)
        q = t % (COUT // 16)
        zbuf[r, pl.ds(q * 16, 16)] = zeros
        return carry

    lax.fori_loop(0, ZB * (COUT // 16), _zfill, 0)
    base_row = s * ROWS_PER_SUB
    for q in range(ROWS_PER_SUB // ZB):
        pltpu.sync_copy(zbuf, hsh.at[pl.ds(base_row + q * ZB, ZB)])

    ebase = wid * EPT
    pltpu.sync_copy(src_hbm.at[pl.ds(ebase, EPT)], src_v)
    pltpu.sync_copy(dst_hbm.at[pl.ds(ebase, EPT)], dst_v)
    pltpu.sync_copy(kidx_hbm.at[pl.ds(ebase, EPT)], kidx_v)

    plsc.subcore_barrier()

    def _chunk(j, carry):
        off = j * CH

        def _build(i, carry2):
            sl = pl.ds(off + i * 16, 16)
            dl = pl.ds(i * 16, 16)
            row_i[dl] = kidx_v[sl] * N + src_v[sl]
            dst_i[dl] = dst_v[sl]
            return carry2

        lax.fori_loop(0, CH // 16, _build, 0)
        pltpu.async_copy(z_hbm.at[row_i], rows_v, sem).wait()
        pltpu.sync_copy(rows_v, hsh.at[dst_i], add=True)
        return carry

    lax.fori_loop(0, NCHUNK, _chunk, 0)

    plsc.subcore_barrier()

    pltpu.sync_copy(hsh.at[pl.ds(base_row, ROWS_PER_SUB)],
                    out_hbm.at[pl.ds(c * NPAD + base_row, ROWS_PER_SUB)])


@functools.cache
def _sc_edges():
    return pl.kernel(
        _sc_body,
        mesh=plsc.VectorSubcoreMesh(core_axis_name="c", subcore_axis_name="s"),
        out_type=jax.ShapeDtypeStruct((NC * NPAD, COUT), jnp.float32),
        scratch_types=[
            pltpu.VMEM((EPT,), jnp.int32),
            pltpu.VMEM((EPT,), jnp.int32),
            pltpu.VMEM((EPT,), jnp.int32),
            pltpu.VMEM((CH,), jnp.int32),
            pltpu.VMEM((CH,), jnp.int32),
            pltpu.VMEM((CH, COUT), jnp.float32),
            pltpu.VMEM((ZB, COUT), jnp.float32),
            pltpu.VMEM_SHARED((NPAD, COUT), jnp.float32),
            pltpu.SemaphoreType.DMA,
        ],
    )


def _bn_body(hp_ref, g_ref, b_ref, o_ref):
    h = hp_ref[:N, :] + hp_ref[NPAD:NPAD + N, :]
    mean = jnp.mean(h, axis=0, keepdims=True)
    hc = h - mean
    var = jnp.mean(hc * hc, axis=0, keepdims=True)
    y = g_ref[...] * (hc * lax.rsqrt(var + EPS)) + b_ref[...]
    o_ref[...] = jnp.maximum(y, 0.0)


def _bn_relu(hp, gamma, beta):
    return pl.pallas_call(
        _bn_body,
        in_specs=[
            pl.BlockSpec((NC * NPAD, COUT), lambda: (0, 0)),
            pl.BlockSpec((1, COUT), lambda: (0, 0)),
            pl.BlockSpec((1, COUT), lambda: (0, 0)),
        ],
        out_specs=pl.BlockSpec((N, COUT), lambda: (0, 0)),
        out_shape=jax.ShapeDtypeStruct((N, COUT), jnp.float32),
    )(hp, gamma, beta)


def kernel(x, edge_index, kernel_idx, W, gamma, beta):
    src = edge_index[0]
    dst = edge_index[1]
    Z = _z_matmul(x, W)
    Z2 = Z.reshape(K * N, COUT)
    hp = _sc_edges()(Z2, src, dst, kernel_idx)
    return _bn_relu(hp, gamma.reshape(1, COUT), beta.reshape(1, COUT))
